# Initial kernel scaffold; baseline (speedup 1.0000x reference)
import jax
import jax.numpy as jnp
from jax.experimental import pallas as pl


def kernel(user_emb, item_emb, edge_index, edge_weight):
    raise NotImplementedError("write your pallas kernel here")


# speedup vs baseline: 1.7946x; 1.7946x over previous
import functools

import jax
import jax.numpy as jnp
from jax import lax
from jax.experimental import pallas as pl
from jax.experimental.pallas import tpu as pltpu
import jax.experimental.pallas.tpu_sc as plsc

D = 64
H_REAL = 25000
H_PAD = 25088
N_PAD = 2 * H_PAD
TRASH = H_REAL
NE = 800000
NE_PAD = 819200
EPT = NE_PAD // 16
W_WIN = 1024
N_WIN = EPT // W_WIN
K = 128
N_CHUNK = W_WIN // K
RPT = H_PAD // 16
EB = 56
L = 16


def _layer_body(final, table, gsrc, dlb, w, sum_in, table_out, sum_out,
                gidx, dloc, w_v, rows, tbuf, sbuf, acc, gsem):
    c = lax.axis_index("c")
    s = lax.axis_index("s")

    zeros = jnp.zeros((L,), jnp.float32)

    def zrow(i, _):
        r = i // 4
        q = (i % 4) * L
        tbuf[r, pl.ds(q, L)] = zeros
        return 0

    lax.fori_loop(0, EB * 4, zrow, 0)

    def zacc(k, _):
        pltpu.sync_copy(tbuf, acc.at[pl.ds(s * RPT + k * EB, EB)])
        return 0

    lax.fori_loop(0, RPT // EB, zacc, 0)
    plsc.subcore_barrier()

    def window(i, _):
        row0 = (s * EPT + i * W_WIN) // K
        base = s * EPT + i * W_WIN
        pltpu.sync_copy(gsrc.at[pl.ds(row0, N_CHUNK)], gidx)
        pltpu.sync_copy(dlb.at[c, pl.ds(row0, N_CHUNK)], dloc)
        pltpu.sync_copy(w.at[pl.ds(base, W_WIN)], w_v)

        for j in range(N_CHUNK):
            pltpu.async_copy(table.at[gidx.at[j]], rows, gsem).wait()

            def scale(g, _):
                wv = w_v[pl.ds(j * K + g * L, L)]
                for ee in range(L):
                    e = g * L + ee
                    wb = jnp.full((L,), wv[ee], jnp.float32)
                    for q in range(4):
                        rows[e, pl.ds(q * L, L)] = rows[e, pl.ds(q * L, L)] * wb
                return 0

            lax.fori_loop(0, K // L, scale, 0)
            pltpu.sync_copy(rows, acc.at[dloc.at[j]], add=True)
        return 0

    lax.fori_loop(0, N_WIN, window, 0)
    plsc.subcore_barrier()

    def addrow(i, _):
        r = i // 4
        q = (i % 4) * L
        v = tbuf[r, pl.ds(q, L)] + sbuf[r, pl.ds(q, L)]
        if final:
            v = v * 0.25
        sbuf[r, pl.ds(q, L)] = v
        return 0

    def epil(k, _):
        r0 = s * RPT + k * EB
        out0 = c * H_PAD + r0
        pltpu.sync_copy(acc.at[pl.ds(r0, EB)], tbuf)
        pltpu.sync_copy(sum_in.at[pl.ds(out0, EB)], sbuf)
        lax.fori_loop(0, EB * 4, addrow, 0)
        pltpu.sync_copy(tbuf, table_out.at[pl.ds(out0, EB)])
        pltpu.sync_copy(sbuf, sum_out.at[pl.ds(out0, EB)])
        return 0

    lax.fori_loop(0, RPT // EB, epil, 0)


def _make_layer(final):
    return pl.kernel(
        functools.partial(_layer_body, final),
        out_type=[
            jax.ShapeDtypeStruct((N_PAD, D), jnp.float32),
            jax.ShapeDtypeStruct((N_PAD, D), jnp.float32),
        ],
        mesh=plsc.VectorSubcoreMesh(core_axis_name="c", subcore_axis_name="s"),
        compiler_params=pltpu.CompilerParams(use_tc_tiling_on_sc=False),
        scratch_types=[
            pltpu.VMEM((N_CHUNK, K), jnp.int32),
            pltpu.VMEM((N_CHUNK, K), jnp.int32),
            pltpu.VMEM((W_WIN,), jnp.float32),
            pltpu.VMEM((K, D), jnp.float32),
            pltpu.VMEM((EB, D), jnp.float32),
            pltpu.VMEM((EB, D), jnp.float32),
            pltpu.VMEM_SHARED((H_PAD, D), jnp.float32),
            pltpu.SemaphoreType.DMA,
        ],
    )


_layer_mid = _make_layer(False)
_layer_fin = _make_layer(True)


def kernel(user_emb, item_emb, edge_index, edge_weight):
    src = edge_index[0].astype(jnp.int32)
    dst = edge_index[1].astype(jnp.int32)
    pad = NE_PAD - NE
    src = jnp.concatenate([src, jnp.zeros((pad,), jnp.int32)])
    dst = jnp.concatenate([dst, jnp.full((pad,), -1, jnp.int32)])
    w = jnp.concatenate([edge_weight.astype(jnp.float32),
                         jnp.zeros((pad,), jnp.float32)])
    gsrc = (src + jnp.where(src >= H_REAL, H_PAD - H_REAL, 0)).reshape(-1, K)
    ok0 = (dst >= 0) & (dst < H_REAL)
    dl0 = jnp.where(ok0, dst, TRASH).reshape(-1, K)
    d1 = dst - H_REAL
    ok1 = (d1 >= 0) & (d1 < H_REAL)
    dl1 = jnp.where(ok1, d1, TRASH).reshape(-1, K)
    dlb = jnp.stack([dl0, dl1])
    z = jnp.zeros((H_PAD - H_REAL, D), jnp.float32)
    table = jnp.concatenate([user_emb, z, item_emb, z], axis=0)
    acc_sum = table
    for li in range(3):
        fn = _layer_fin if li == 2 else _layer_mid
        table, acc_sum = fn(table, gsrc, dlb, w, acc_sum)
    users = acc_sum[:H_REAL]
    items = acc_sum[H_PAD:H_PAD + H_REAL]
    return (users, items)


# speedup vs baseline: 2.1700x; 1.2092x over previous
import functools

import jax
import jax.numpy as jnp
from jax import lax
from jax.experimental import pallas as pl
from jax.experimental.pallas import tpu as pltpu
import jax.experimental.pallas.tpu_sc as plsc

D = 64
H_REAL = 25000
H_PAD = 25088
N_PAD = 2 * H_PAD
TRASH = H_REAL
NE = 800000
NE_PAD = 819200
EPT = NE_PAD // 16
W_WIN = 1024
N_WIN = EPT // W_WIN
K = 128
N_CHUNK = W_WIN // K
RPT = H_PAD // 16
EB = 56
L = 16


def _layer_body(final, table, gsrc, dlb, w, sum_in, table_out, sum_out,
                gidx, dloc, w_v, rows0, rows1, tbuf, sbuf, acc,
                gsem0, gsem1, ssem0, ssem1):
    c = lax.axis_index("c")
    s = lax.axis_index("s")

    zeros = jnp.zeros((L,), jnp.float32)

    def zrow(i, _):
        r = i // 4
        q = (i % 4) * L
        tbuf[r, pl.ds(q, L)] = zeros
        return 0

    lax.fori_loop(0, EB * 4, zrow, 0)

    def zacc(k, _):
        pltpu.sync_copy(tbuf, acc.at[pl.ds(s * RPT + k * EB, EB)])
        return 0

    lax.fori_loop(0, RPT // EB, zacc, 0)
    plsc.subcore_barrier()

    def window(i, _):
        row0 = (s * EPT + i * W_WIN) // K
        base = s * EPT + i * W_WIN
        pltpu.sync_copy(gsrc.at[pl.ds(row0, N_CHUNK)], gidx)
        pltpu.sync_copy(dlb.at[c, pl.ds(row0, N_CHUNK)], dloc)
        pltpu.sync_copy(w.at[pl.ds(base, W_WIN)], w_v)

        rbufs = (rows0, rows1)
        gsems = (gsem0, gsem1)
        ssems = (ssem0, ssem1)
        g_cur = pltpu.async_copy(table.at[gidx.at[0]], rbufs[0], gsems[0])
        s_prev = None
        for j in range(N_CHUNK):
            b = j % 2
            if j < N_CHUNK - 1:
                if s_prev is not None:
                    s_prev.wait()
                    s_prev = None
                g_next = pltpu.async_copy(
                    table.at[gidx.at[j + 1]], rbufs[1 - b], gsems[1 - b])
            g_cur.wait()
            buf = rbufs[b]

            def scale(g, _):
                wv = w_v[pl.ds(j * K + g * L, L)]
                for ee in range(L):
                    e = g * L + ee
                    wb = jnp.full((L,), wv[ee], jnp.float32)
                    for q in range(4):
                        buf[e, pl.ds(q * L, L)] = buf[e, pl.ds(q * L, L)] * wb
                return 0

            lax.fori_loop(0, K // L, scale, 0)
            s_cur = pltpu.async_copy(buf, acc.at[dloc.at[j]], ssems[b], add=True)
            if j < N_CHUNK - 1:
                s_prev, g_cur = s_cur, g_next
            else:
                if s_prev is not None:
                    s_prev.wait()
                s_cur.wait()
        return 0

    lax.fori_loop(0, N_WIN, window, 0)
    plsc.subcore_barrier()

    def addrow(i, _):
        r = i // 4
        q = (i % 4) * L
        v = tbuf[r, pl.ds(q, L)] + sbuf[r, pl.ds(q, L)]
        if final:
            v = v * 0.25
        sbuf[r, pl.ds(q, L)] = v
        return 0

    def epil(k, _):
        r0 = s * RPT + k * EB
        out0 = c * H_PAD + r0
        pltpu.sync_copy(acc.at[pl.ds(r0, EB)], tbuf)
        pltpu.sync_copy(sum_in.at[pl.ds(out0, EB)], sbuf)
        lax.fori_loop(0, EB * 4, addrow, 0)
        pltpu.sync_copy(tbuf, table_out.at[pl.ds(out0, EB)])
        pltpu.sync_copy(sbuf, sum_out.at[pl.ds(out0, EB)])
        return 0

    lax.fori_loop(0, RPT // EB, epil, 0)


def _make_layer(final):
    return pl.kernel(
        functools.partial(_layer_body, final),
        out_type=[
            jax.ShapeDtypeStruct((N_PAD, D), jnp.float32),
            jax.ShapeDtypeStruct((N_PAD, D), jnp.float32),
        ],
        mesh=plsc.VectorSubcoreMesh(core_axis_name="c", subcore_axis_name="s"),
        compiler_params=pltpu.CompilerParams(use_tc_tiling_on_sc=False),
        scratch_types=[
            pltpu.VMEM((N_CHUNK, K), jnp.int32),
            pltpu.VMEM((N_CHUNK, K), jnp.int32),
            pltpu.VMEM((W_WIN,), jnp.float32),
            pltpu.VMEM((K, D), jnp.float32),
            pltpu.VMEM((K, D), jnp.float32),
            pltpu.VMEM((EB, D), jnp.float32),
            pltpu.VMEM((EB, D), jnp.float32),
            pltpu.VMEM_SHARED((H_PAD, D), jnp.float32),
            pltpu.SemaphoreType.DMA,
            pltpu.SemaphoreType.DMA,
            pltpu.SemaphoreType.DMA,
            pltpu.SemaphoreType.DMA,
        ],
    )


_layer_mid = _make_layer(False)
_layer_fin = _make_layer(True)


def kernel(user_emb, item_emb, edge_index, edge_weight):
    src = edge_index[0].astype(jnp.int32)
    dst = edge_index[1].astype(jnp.int32)
    pad = NE_PAD - NE
    src = jnp.concatenate([src, jnp.zeros((pad,), jnp.int32)])
    dst = jnp.concatenate([dst, jnp.full((pad,), -1, jnp.int32)])
    w = jnp.concatenate([edge_weight.astype(jnp.float32),
                         jnp.zeros((pad,), jnp.float32)])
    gsrc = (src + jnp.where(src >= H_REAL, H_PAD - H_REAL, 0)).reshape(-1, K)
    ok0 = (dst >= 0) & (dst < H_REAL)
    dl0 = jnp.where(ok0, dst, TRASH).reshape(-1, K)
    d1 = dst - H_REAL
    ok1 = (d1 >= 0) & (d1 < H_REAL)
    dl1 = jnp.where(ok1, d1, TRASH).reshape(-1, K)
    dlb = jnp.stack([dl0, dl1])
    z = jnp.zeros((H_PAD - H_REAL, D), jnp.float32)
    table = jnp.concatenate([user_emb, z, item_emb, z], axis=0)
    acc_sum = table
    for li in range(3):
        fn = _layer_fin if li == 2 else _layer_mid
        table, acc_sum = fn(table, gsrc, dlb, w, acc_sum)
    users = acc_sum[:H_REAL]
    items = acc_sum[H_PAD:H_PAD + H_REAL]
    return (users, items)


# speedup vs baseline: 2.7426x; 1.2639x over previous
import functools

import jax
import jax.numpy as jnp
from jax import lax
from jax.experimental import pallas as pl
from jax.experimental.pallas import tpu as pltpu
import jax.experimental.pallas.tpu_sc as plsc

D = 64
H_REAL = 25000
H_PAD = 25088
N_PAD = 2 * H_PAD
TRASH = H_REAL
NE = 800000
NE_PAD = 819200
SCAN = NE_PAD // 32
W_WIN = 1024
N_SCAN_WIN = SCAN // W_WIN
R_CAP = SCAN + W_WIN
K = 128
N_CHUNK = W_WIN // K
RPT = H_PAD // 16
EB = 56
L = 16


def _partition_body(gsrc, dl0, dl1, w, pg, pd, pw, cnt,
                    in_g, in_d0, in_d1, in_w, ob_g, ob_d, ob_w, cbuf):
    c = lax.axis_index("c")
    s = lax.axis_index("s")
    wid = s * 2 + c

    for half in range(2):
        in_d = (in_d0, in_d1)[half]

        tvec0 = jnp.full((L,), TRASH, jnp.int32)
        zvec0 = jnp.zeros((L,), jnp.float32)
        gvec0 = jnp.zeros((L,), jnp.int32)

        def prefill(v, _):
            ob_g[pl.ds(v * L, L)] = gvec0
            ob_d[pl.ds(v * L, L)] = tvec0
            ob_w[pl.ds(v * L, L)] = zvec0
            return 0

        lax.fori_loop(0, (R_CAP + L) // L, prefill, 0)

        def win(i, off):
            base = wid * SCAN + i * W_WIN
            pltpu.sync_copy(gsrc.at[pl.ds(base, W_WIN)], in_g)
            pltpu.sync_copy(dl0.at[pl.ds(base, W_WIN)], in_d0)
            pltpu.sync_copy(dl1.at[pl.ds(base, W_WIN)], in_d1)
            pltpu.sync_copy(w.at[pl.ds(base, W_WIN)], in_w)

            def vec(v, off):
                dv = in_d[pl.ds(v * L, L)]
                gv = in_g[pl.ds(v * L, L)]
                wv = in_w[pl.ds(v * L, L)]
                m = dv != TRASH
                incl = plsc.cumsum(m.astype(jnp.int32))
                lane = lax.iota(jnp.int32, L)
                pos = jnp.where(m, off + incl - 1, R_CAP + lane)
                plsc.store_scatter(ob_g, [pos], gv)
                plsc.store_scatter(ob_d, [pos], dv)
                plsc.store_scatter(ob_w, [pos], wv)
                return off + incl[L - 1]

            return lax.fori_loop(0, W_WIN // L, vec, off)

        off = lax.fori_loop(0, N_SCAN_WIN, win, jnp.int32(0))

        n_win = (off + W_WIN - 1) // W_WIN
        pad_end = n_win * W_WIN
        tvec = jnp.full((L,), TRASH, jnp.int32)
        zvec = jnp.zeros((L,), jnp.float32)

        def padv(v, off):
            ob_g[pl.ds(off, L)] = tvec
            ob_d[pl.ds(off, L)] = tvec
            ob_w[pl.ds(off, L)] = zvec
            return off + L

        lax.fori_loop(0, (pad_end - off) // L, padv, off)

        pltpu.sync_copy(ob_g.at[pl.ds(0, R_CAP)], pg.at[half, wid])
        pltpu.sync_copy(ob_d.at[pl.ds(0, R_CAP)], pd.at[half, wid])
        pltpu.sync_copy(ob_w.at[pl.ds(0, R_CAP)], pw.at[half, wid])
        cbuf[pl.ds(0, L)] = jnp.full((L,), n_win, jnp.int32)
        pltpu.sync_copy(cbuf, cnt.at[half, wid])


_partition = pl.kernel(
    _partition_body,
    out_type=[
        jax.ShapeDtypeStruct((2, 32, R_CAP), jnp.int32),
        jax.ShapeDtypeStruct((2, 32, R_CAP), jnp.int32),
        jax.ShapeDtypeStruct((2, 32, R_CAP), jnp.float32),
        jax.ShapeDtypeStruct((2, 32, L), jnp.int32),
    ],
    mesh=plsc.VectorSubcoreMesh(core_axis_name="c", subcore_axis_name="s"),
    compiler_params=pltpu.CompilerParams(use_tc_tiling_on_sc=False,
                                         needs_layout_passes=False),
    scratch_types=[
        pltpu.VMEM((W_WIN,), jnp.int32),
        pltpu.VMEM((W_WIN,), jnp.int32),
        pltpu.VMEM((W_WIN,), jnp.int32),
        pltpu.VMEM((W_WIN,), jnp.float32),
        pltpu.VMEM((R_CAP + L,), jnp.int32),
        pltpu.VMEM((R_CAP + L,), jnp.int32),
        pltpu.VMEM((R_CAP + L,), jnp.float32),
        pltpu.VMEM((L,), jnp.int32),
    ],
)


def _layer_body(final, table, pg, pd, pw, cnt, sum_in, table_out, sum_out,
                gidx, dloc, w_v, cnts, rows0, rows1, tbuf, sbuf, acc,
                gsem0, gsem1, ssem0, ssem1):
    c = lax.axis_index("c")
    s = lax.axis_index("s")

    zeros = jnp.zeros((L,), jnp.float32)

    def zrow(i, _):
        r = i // 4
        q = (i % 4) * L
        tbuf[r, pl.ds(q, L)] = zeros
        return 0

    lax.fori_loop(0, EB * 4, zrow, 0)

    def zacc(k, _):
        pltpu.sync_copy(tbuf, acc.at[pl.ds(s * RPT + k * EB, EB)])
        return 0

    lax.fori_loop(0, RPT // EB, zacc, 0)
    pltpu.sync_copy(cnt.at[c], cnts)
    plsc.subcore_barrier()

    def window(rr, i):
        pltpu.sync_copy(pg.at[c, rr, pl.ds(i * N_CHUNK, N_CHUNK)], gidx)
        pltpu.sync_copy(pd.at[c, rr, pl.ds(i * N_CHUNK, N_CHUNK)], dloc)
        pltpu.sync_copy(pw.at[c, rr, pl.ds(i * W_WIN, W_WIN)], w_v)

        rbufs = (rows0, rows1)
        gsems = (gsem0, gsem1)
        ssems = (ssem0, ssem1)
        g_cur = pltpu.async_copy(table.at[gidx.at[0]], rbufs[0], gsems[0])
        s_prev = None
        for j in range(N_CHUNK):
            b = j % 2
            if j < N_CHUNK - 1:
                if s_prev is not None:
                    s_prev.wait()
                    s_prev = None
                g_next = pltpu.async_copy(
                    table.at[gidx.at[j + 1]], rbufs[1 - b], gsems[1 - b])
            g_cur.wait()
            buf = rbufs[b]

            def scale(g, _):
                wv = w_v[pl.ds(j * K + g * L, L)]
                for ee in range(L):
                    e = g * L + ee
                    wb = jnp.full((L,), wv[ee], jnp.float32)
                    for q in range(4):
                        buf[e, pl.ds(q * L, L)] = buf[e, pl.ds(q * L, L)] * wb
                return 0

            lax.fori_loop(0, K // L, scale, 0)
            s_cur = pltpu.async_copy(buf, acc.at[dloc.at[j]], ssems[b], add=True)
            if j < N_CHUNK - 1:
                s_prev, g_cur = s_cur, g_next
            else:
                if s_prev is not None:
                    s_prev.wait()
                s_cur.wait()

    for r in range(2):
        rr = s * 2 + r
        nv = cnts[rr, pl.ds(0, L)]
        n_win = jnp.minimum(nv[0], R_CAP // W_WIN)

        def wbody(i, _):
            window(rr, i)
            return 0

        lax.fori_loop(0, n_win, wbody, 0)

    plsc.subcore_barrier()

    def addrow(i, _):
        r = i // 4
        q = (i % 4) * L
        v = tbuf[r, pl.ds(q, L)] + sbuf[r, pl.ds(q, L)]
        if final:
            v = v * 0.25
        sbuf[r, pl.ds(q, L)] = v
        return 0

    def epil(k, _):
        r0 = s * RPT + k * EB
        out0 = c * H_PAD + r0
        pltpu.sync_copy(acc.at[pl.ds(r0, EB)], tbuf)
        pltpu.sync_copy(sum_in.at[pl.ds(out0, EB)], sbuf)
        lax.fori_loop(0, EB * 4, addrow, 0)
        pltpu.sync_copy(tbuf, table_out.at[pl.ds(out0, EB)])
        pltpu.sync_copy(sbuf, sum_out.at[pl.ds(out0, EB)])
        return 0

    lax.fori_loop(0, RPT // EB, epil, 0)


def _make_layer(final):
    return pl.kernel(
        functools.partial(_layer_body, final),
        out_type=[
            jax.ShapeDtypeStruct((N_PAD, D), jnp.float32),
            jax.ShapeDtypeStruct((N_PAD, D), jnp.float32),
        ],
        mesh=plsc.VectorSubcoreMesh(core_axis_name="c", subcore_axis_name="s"),
        compiler_params=pltpu.CompilerParams(use_tc_tiling_on_sc=False),
        scratch_types=[
            pltpu.VMEM((N_CHUNK, K), jnp.int32),
            pltpu.VMEM((N_CHUNK, K), jnp.int32),
            pltpu.VMEM((W_WIN,), jnp.float32),
            pltpu.VMEM((32, L), jnp.int32),
            pltpu.VMEM((K, D), jnp.float32),
            pltpu.VMEM((K, D), jnp.float32),
            pltpu.VMEM((EB, D), jnp.float32),
            pltpu.VMEM((EB, D), jnp.float32),
            pltpu.VMEM_SHARED((H_PAD, D), jnp.float32),
            pltpu.SemaphoreType.DMA,
            pltpu.SemaphoreType.DMA,
            pltpu.SemaphoreType.DMA,
            pltpu.SemaphoreType.DMA,
        ],
    )


_layer_mid = _make_layer(False)
_layer_fin = _make_layer(True)


def kernel(user_emb, item_emb, edge_index, edge_weight):
    src = edge_index[0].astype(jnp.int32)
    dst = edge_index[1].astype(jnp.int32)
    pad = NE_PAD - NE
    src = jnp.concatenate([src, jnp.zeros((pad,), jnp.int32)])
    dst = jnp.concatenate([dst, jnp.full((pad,), -1, jnp.int32)])
    w = jnp.concatenate([edge_weight.astype(jnp.float32),
                         jnp.zeros((pad,), jnp.float32)])
    gsrc = src + jnp.where(src >= H_REAL, H_PAD - H_REAL, 0)
    ok0 = (dst >= 0) & (dst < H_REAL)
    dl0 = jnp.where(ok0, dst, TRASH)
    d1 = dst - H_REAL
    ok1 = (d1 >= 0) & (d1 < H_REAL)
    dl1 = jnp.where(ok1, d1, TRASH)
    z = jnp.zeros((H_PAD - H_REAL, D), jnp.float32)
    table = jnp.concatenate([user_emb, z, item_emb, z], axis=0)

    pg, pd, pw, cnt = _partition(gsrc, dl0, dl1, w)
    pg = pg.reshape(2, 32, R_CAP // K, K)
    pd = pd.reshape(2, 32, R_CAP // K, K)

    acc_sum = table
    for li in range(3):
        fn = _layer_fin if li == 2 else _layer_mid
        table, acc_sum = fn(table, pg, pd, pw, cnt, acc_sum)
    users = acc_sum[:H_REAL]
    items = acc_sum[H_PAD:H_PAD + H_REAL]
    return (users, items)


# speedup vs baseline: 2.8007x; 1.0212x over previous
import functools

import jax
import jax.numpy as jnp
from jax import lax
from jax.experimental import pallas as pl
from jax.experimental.pallas import tpu as pltpu
import jax.experimental.pallas.tpu_sc as plsc

D = 64
H_REAL = 25000
H_PAD = 25088
N_PAD = 2 * H_PAD
TRASH = H_REAL
NE = 800000
NE_PAD = 819200
SCAN = NE_PAD // 32
W_WIN = 1024
N_SCAN_WIN = SCAN // W_WIN
R_CAP = SCAN + W_WIN
K = 128
N_CHUNK = W_WIN // K
RPT = H_PAD // 16
EB = 28
L = 16


def _partition_body(gsrc, dl0, dl1, w, pg, pd, pw, cnt,
                    in_g, in_d0, in_d1, in_w, ob_g, ob_d, ob_w, cbuf):
    c = lax.axis_index("c")
    s = lax.axis_index("s")
    wid = s * 2 + c

    for half in range(2):
        in_d = (in_d0, in_d1)[half]

        tvec0 = jnp.full((L,), TRASH, jnp.int32)
        zvec0 = jnp.zeros((L,), jnp.float32)
        gvec0 = jnp.zeros((L,), jnp.int32)

        def prefill(v, _):
            ob_g[pl.ds(v * L, L)] = gvec0
            ob_d[pl.ds(v * L, L)] = tvec0
            ob_w[pl.ds(v * L, L)] = zvec0
            return 0

        lax.fori_loop(0, (R_CAP + L) // L, prefill, 0)

        def win(i, off):
            base = wid * SCAN + i * W_WIN
            pltpu.sync_copy(gsrc.at[pl.ds(base, W_WIN)], in_g)
            pltpu.sync_copy(dl0.at[pl.ds(base, W_WIN)], in_d0)
            pltpu.sync_copy(dl1.at[pl.ds(base, W_WIN)], in_d1)
            pltpu.sync_copy(w.at[pl.ds(base, W_WIN)], in_w)

            def vec(v, off):
                dv = in_d[pl.ds(v * L, L)]
                gv = in_g[pl.ds(v * L, L)]
                wv = in_w[pl.ds(v * L, L)]
                m = dv != TRASH
                incl = plsc.cumsum(m.astype(jnp.int32))
                lane = lax.iota(jnp.int32, L)
                pos = jnp.where(m, off + incl - 1, R_CAP + lane)
                plsc.store_scatter(ob_g, [pos], gv)
                plsc.store_scatter(ob_d, [pos], dv)
                plsc.store_scatter(ob_w, [pos], wv)
                return off + incl[L - 1]

            return lax.fori_loop(0, W_WIN // L, vec, off)

        off = lax.fori_loop(0, N_SCAN_WIN, win, jnp.int32(0))

        n_win = (off + W_WIN - 1) // W_WIN
        pad_end = n_win * W_WIN
        tvec = jnp.full((L,), TRASH, jnp.int32)
        zvec = jnp.zeros((L,), jnp.float32)

        def padv(v, off):
            ob_g[pl.ds(off, L)] = tvec
            ob_d[pl.ds(off, L)] = tvec
            ob_w[pl.ds(off, L)] = zvec
            return off + L

        lax.fori_loop(0, (pad_end - off) // L, padv, off)

        pltpu.sync_copy(ob_g.at[pl.ds(0, R_CAP)], pg.at[half, wid])
        pltpu.sync_copy(ob_d.at[pl.ds(0, R_CAP)], pd.at[half, wid])
        pltpu.sync_copy(ob_w.at[pl.ds(0, R_CAP)], pw.at[half, wid])
        cbuf[pl.ds(0, L)] = jnp.full((L,), n_win, jnp.int32)
        pltpu.sync_copy(cbuf, cnt.at[half, wid])


_partition = pl.kernel(
    _partition_body,
    out_type=[
        jax.ShapeDtypeStruct((2, 32, R_CAP), jnp.int32),
        jax.ShapeDtypeStruct((2, 32, R_CAP), jnp.int32),
        jax.ShapeDtypeStruct((2, 32, R_CAP), jnp.float32),
        jax.ShapeDtypeStruct((2, 32, L), jnp.int32),
    ],
    mesh=plsc.VectorSubcoreMesh(core_axis_name="c", subcore_axis_name="s"),
    compiler_params=pltpu.CompilerParams(use_tc_tiling_on_sc=False,
                                         needs_layout_passes=False),
    scratch_types=[
        pltpu.VMEM((W_WIN,), jnp.int32),
        pltpu.VMEM((W_WIN,), jnp.int32),
        pltpu.VMEM((W_WIN,), jnp.int32),
        pltpu.VMEM((W_WIN,), jnp.float32),
        pltpu.VMEM((R_CAP + L,), jnp.int32),
        pltpu.VMEM((R_CAP + L,), jnp.int32),
        pltpu.VMEM((R_CAP + L,), jnp.float32),
        pltpu.VMEM((L,), jnp.int32),
    ],
)


def _layer_body(final, table, pg, pd, pw, cnt, sum_in, table_out, sum_out,
                gidx, dloc, w_v, cnts, rows0, rows1, rows2, acc,
                gsem0, gsem1, gsem2, ssem0, ssem1, ssem2):
    tbuf = rows0
    sbuf = rows1
    c = lax.axis_index("c")
    s = lax.axis_index("s")

    zeros = jnp.zeros((L,), jnp.float32)

    def zrow(i, _):
        r = i // 4
        q = (i % 4) * L
        tbuf[r, pl.ds(q, L)] = zeros
        return 0

    lax.fori_loop(0, EB * 4, zrow, 0)

    def zacc(k, _):
        pltpu.sync_copy(tbuf.at[pl.ds(0, EB)], acc.at[pl.ds(s * RPT + k * EB, EB)])
        return 0

    lax.fori_loop(0, RPT // EB, zacc, 0)
    pltpu.sync_copy(cnt.at[c], cnts)
    plsc.subcore_barrier()

    def window(rr, i):
        pltpu.sync_copy(pg.at[c, rr, pl.ds(i * N_CHUNK, N_CHUNK)], gidx)
        pltpu.sync_copy(pd.at[c, rr, pl.ds(i * N_CHUNK, N_CHUNK)], dloc)
        pltpu.sync_copy(pw.at[c, rr, pl.ds(i * W_WIN, W_WIN)], w_v)

        rbufs = (rows0, rows1, rows2)
        gsems = (gsem0, gsem1, gsem2)
        ssems = (ssem0, ssem1, ssem2)
        NB = 3
        gd = [None] * N_CHUNK
        sd = [None] * N_CHUNK
        gd[0] = pltpu.async_copy(table.at[gidx.at[0]], rbufs[0], gsems[0])
        for j in range(N_CHUNK):
            b = j % NB
            if j < N_CHUNK - 1:
                nb = (j + 1) % NB
                if j + 1 >= NB and sd[j + 1 - NB] is not None:
                    sd[j + 1 - NB].wait()
                    sd[j + 1 - NB] = None
                gd[j + 1] = pltpu.async_copy(
                    table.at[gidx.at[j + 1]], rbufs[nb], gsems[nb])
            gd[j].wait()
            buf = rbufs[b]

            def scale(g, _):
                wv = w_v[pl.ds(j * K + g * L, L)]
                for ee in range(L):
                    e = g * L + ee
                    wb = jnp.full((L,), wv[ee], jnp.float32)
                    for q in range(4):
                        buf[e, pl.ds(q * L, L)] = buf[e, pl.ds(q * L, L)] * wb
                return 0

            lax.fori_loop(0, K // L, scale, 0)
            sd[j] = pltpu.async_copy(buf, acc.at[dloc.at[j]], ssems[b], add=True)
        for d in sd:
            if d is not None:
                d.wait()

    for r in range(2):
        rr = s * 2 + r
        nv = cnts[rr, pl.ds(0, L)]
        n_win = jnp.minimum(nv[0], R_CAP // W_WIN)

        def wbody(i, _):
            window(rr, i)
            return 0

        lax.fori_loop(0, n_win, wbody, 0)

    plsc.subcore_barrier()

    def addrow(i, _):
        r = i // 4
        q = (i % 4) * L
        v = tbuf[r, pl.ds(q, L)] + sbuf[r, pl.ds(q, L)]
        if final:
            v = v * 0.25
        sbuf[r, pl.ds(q, L)] = v
        return 0

    def epil(k, _):
        r0 = s * RPT + k * EB
        out0 = c * H_PAD + r0
        pltpu.sync_copy(acc.at[pl.ds(r0, EB)], tbuf.at[pl.ds(0, EB)])
        pltpu.sync_copy(sum_in.at[pl.ds(out0, EB)], sbuf.at[pl.ds(0, EB)])
        lax.fori_loop(0, EB * 4, addrow, 0)
        pltpu.sync_copy(tbuf.at[pl.ds(0, EB)], table_out.at[pl.ds(out0, EB)])
        pltpu.sync_copy(sbuf.at[pl.ds(0, EB)], sum_out.at[pl.ds(out0, EB)])
        return 0

    lax.fori_loop(0, RPT // EB, epil, 0)


def _make_layer(final):
    return pl.kernel(
        functools.partial(_layer_body, final),
        out_type=[
            jax.ShapeDtypeStruct((N_PAD, D), jnp.float32),
            jax.ShapeDtypeStruct((N_PAD, D), jnp.float32),
        ],
        mesh=plsc.VectorSubcoreMesh(core_axis_name="c", subcore_axis_name="s"),
        compiler_params=pltpu.CompilerParams(use_tc_tiling_on_sc=False),
        scratch_types=[
            pltpu.VMEM((N_CHUNK, K), jnp.int32),
            pltpu.VMEM((N_CHUNK, K), jnp.int32),
            pltpu.VMEM((W_WIN,), jnp.float32),
            pltpu.VMEM((32, L), jnp.int32),
            pltpu.VMEM((K, D), jnp.float32),
            pltpu.VMEM((K, D), jnp.float32),
            pltpu.VMEM((K, D), jnp.float32),
            pltpu.VMEM_SHARED((H_PAD, D), jnp.float32),
            pltpu.SemaphoreType.DMA,
            pltpu.SemaphoreType.DMA,
            pltpu.SemaphoreType.DMA,
            pltpu.SemaphoreType.DMA,
            pltpu.SemaphoreType.DMA,
            pltpu.SemaphoreType.DMA,
        ],
    )


_layer_mid = _make_layer(False)
_layer_fin = _make_layer(True)


def kernel(user_emb, item_emb, edge_index, edge_weight):
    src = edge_index[0].astype(jnp.int32)
    dst = edge_index[1].astype(jnp.int32)
    pad = NE_PAD - NE
    src = jnp.concatenate([src, jnp.zeros((pad,), jnp.int32)])
    dst = jnp.concatenate([dst, jnp.full((pad,), -1, jnp.int32)])
    w = jnp.concatenate([edge_weight.astype(jnp.float32),
                         jnp.zeros((pad,), jnp.float32)])
    gsrc = src + jnp.where(src >= H_REAL, H_PAD - H_REAL, 0)
    ok0 = (dst >= 0) & (dst < H_REAL)
    dl0 = jnp.where(ok0, dst, TRASH)
    d1 = dst - H_REAL
    ok1 = (d1 >= 0) & (d1 < H_REAL)
    dl1 = jnp.where(ok1, d1, TRASH)
    z = jnp.zeros((H_PAD - H_REAL, D), jnp.float32)
    table = jnp.concatenate([user_emb, z, item_emb, z], axis=0)

    pg, pd, pw, cnt = _partition(gsrc, dl0, dl1, w)
    pg = pg.reshape(2, 32, R_CAP // K, K)
    pd = pd.reshape(2, 32, R_CAP // K, K)

    acc_sum = table
    for li in range(3):
        fn = _layer_fin if li == 2 else _layer_mid
        table, acc_sum = fn(table, pg, pd, pw, cnt, acc_sum)
    users = acc_sum[:H_REAL]
    items = acc_sum[H_PAD:H_PAD + H_REAL]
    return (users, items)
